# baseline (speedup 1.0000x reference)
import dataclasses

import jax
import jax.numpy as jnp
from jax import lax
from jax.experimental import pallas as pl
from jax.experimental.pallas import tpu as pltpu
from jax.experimental.pallas import tpu_sc as plsc

B = 4
C = 96
K = 1024
N = 512 * 512

CG = 4
CH = C // CG
SG = 2
NS = N // SG
P = 1024
T = NS // P

_mesh = plsc.VectorSubcoreMesh(core_axis_name="c", subcore_axis_name="s")

_cp = pltpu.CompilerParams()
if "needs_layout_passes" in pltpu.CompilerParams.__dataclass_fields__:
    _cp = dataclasses.replace(_cp, needs_layout_passes=False)


def _unpool_body(pooled_hbm, spx_hbm, out_hbm, table_v,
                 idx_v, out_t, sem_i, sem_o):
    wid = lax.axis_index("c") * 16 + lax.axis_index("s")
    b = wid // (CG * SG)
    rem = wid % (CG * SG)
    cg = rem // SG
    sg = rem % SG

    tbl_off = (b * C + cg * CH) * K
    pltpu.sync_copy(pooled_hbm.at[pl.ds(tbl_off, CH * K)], table_v)

    idx_base = b * N + sg * NS

    def start_idx(t, buf):
        pltpu.async_copy(spx_hbm.at[pl.ds(idx_base + t * P, P)],
                         idx_v.at[buf], sem_i[buf])

    def item(t, buf):
        pltpu.make_async_copy(spx_hbm.at[pl.ds(idx_base, P)],
                              idx_v.at[buf], sem_i[buf]).wait()

        @pl.when(t >= 2)
        def _():
            pltpu.make_async_copy(
                out_t.at[buf],
                out_hbm.at[b, pl.ds(cg * CH, CH), pl.ds(idx_base, P)],
                sem_o[buf]).wait()

        @plsc.parallel_loop(0, P, 16, unroll=2)
        def _vec(i):
            idx = idx_v[buf, pl.ds(i, 16)]
            for c in range(CH):
                out_t[buf, c, pl.ds(i, 16)] = plsc.load_gather(
                    table_v, [idx + c * K])

        pltpu.async_copy(
            out_t.at[buf],
            out_hbm.at[b, pl.ds(cg * CH, CH),
                       pl.ds(sg * NS + t * P, P)],
            sem_o[buf])

        @pl.when(t + 2 < T)
        def _():
            start_idx(t + 2, buf)

    start_idx(0, 0)
    start_idx(1, 1)

    @pl.loop(0, T, step=2)
    def _tile(t):
        item(t, 0)
        item(t + 1, 1)

    for buf in range(2):
        pltpu.make_async_copy(
            out_t.at[buf],
            out_hbm.at[b, pl.ds(cg * CH, CH), pl.ds(idx_base, P)],
            sem_o[buf]).wait()


@jax.jit
def kernel(pooled, spx):
    spx_i = spx.reshape(B * N).astype(jnp.int32)
    pooled_f = pooled.reshape(B * C * K)

    k = pl.kernel(
        _unpool_body,
        out_type=jax.ShapeDtypeStruct((B, C, N), jnp.float32),
        mesh=_mesh,
        compiler_params=_cp,
        scratch_types=[
            pltpu.VMEM((CH * K,), jnp.float32),
            pltpu.VMEM((2, P), jnp.int32),
            pltpu.VMEM((2, CH, P), jnp.float32),
            [pltpu.SemaphoreType.DMA] * 2,
            [pltpu.SemaphoreType.DMA] * 2,
        ],
    )
    return k(pooled_f, spx_i).reshape(B, C, 512, 512)


# speedup vs baseline: 1.0010x; 1.0010x over previous
import dataclasses

import jax
import jax.numpy as jnp
from jax import lax
from jax.experimental import pallas as pl
from jax.experimental.pallas import tpu as pltpu
from jax.experimental.pallas import tpu_sc as plsc

B = 4
C = 96
K = 1024
N = 512 * 512

CG = 4
CH = C // CG
SG = 2
NS = N // SG
P = 1024
T = NS // P

_mesh = plsc.VectorSubcoreMesh(core_axis_name="c", subcore_axis_name="s")

_cp = pltpu.CompilerParams()
if "needs_layout_passes" in pltpu.CompilerParams.__dataclass_fields__:
    _cp = dataclasses.replace(_cp, needs_layout_passes=False)


def _unpool_body(pooled_hbm, spx_hbm, out_hbm, table_v,
                 idx_v, out_t, sem_i, sem_o):
    wid = lax.axis_index("c") * 16 + lax.axis_index("s")
    b = wid // (CG * SG)
    rem = wid % (CG * SG)
    cg = rem // SG
    sg = rem % SG

    tbl_off = (b * C + cg * CH) * K
    pltpu.sync_copy(pooled_hbm.at[pl.ds(tbl_off, CH * K)], table_v)

    idx_base = b * N + sg * NS

    def start_idx(t, buf):
        pltpu.async_copy(spx_hbm.at[pl.ds(idx_base + t * P, P)],
                         idx_v.at[buf], sem_i[buf])

    def item(t, buf):
        pltpu.make_async_copy(spx_hbm.at[pl.ds(idx_base, P)],
                              idx_v.at[buf], sem_i[buf]).wait()

        @pl.when(t >= 2)
        def _():
            pltpu.make_async_copy(
                out_t.at[buf],
                out_hbm.at[b, pl.ds(cg * CH, CH), pl.ds(idx_base, P)],
                sem_o[buf]).wait()

        @plsc.parallel_loop(0, P, 16, unroll=2)
        def _vec(i):
            idx = idx_v[buf, pl.ds(i, 16)]
            for c in range(CH):
                out_t[buf, c, pl.ds(i, 16)] = plsc.load_gather(
                    table_v.at[pl.ds(c * K, K)], [idx])

        pltpu.async_copy(
            out_t.at[buf],
            out_hbm.at[b, pl.ds(cg * CH, CH),
                       pl.ds(sg * NS + t * P, P)],
            sem_o[buf])

        @pl.when(t + 2 < T)
        def _():
            start_idx(t + 2, buf)

    start_idx(0, 0)
    start_idx(1, 1)

    @pl.loop(0, T, step=2)
    def _tile(t):
        item(t, 0)
        item(t + 1, 1)

    for buf in range(2):
        pltpu.make_async_copy(
            out_t.at[buf],
            out_hbm.at[b, pl.ds(cg * CH, CH), pl.ds(idx_base, P)],
            sem_o[buf]).wait()


@jax.jit
def kernel(pooled, spx):
    spx_i = spx.reshape(B * N).astype(jnp.int32)
    pooled_f = pooled.reshape(B * C * K)

    k = pl.kernel(
        _unpool_body,
        out_type=jax.ShapeDtypeStruct((B, C, N), jnp.float32),
        mesh=_mesh,
        compiler_params=_cp,
        scratch_types=[
            pltpu.VMEM((CH * K,), jnp.float32),
            pltpu.VMEM((2, P), jnp.int32),
            pltpu.VMEM((2, CH, P), jnp.float32),
            [pltpu.SemaphoreType.DMA] * 2,
            [pltpu.SemaphoreType.DMA] * 2,
        ],
    )
    return k(pooled_f, spx_i).reshape(B, C, 512, 512)


# speedup vs baseline: 2.3228x; 2.3206x over previous
import dataclasses

import jax
import jax.numpy as jnp
from jax import lax
from jax.experimental import pallas as pl
from jax.experimental.pallas import tpu as pltpu
from jax.experimental.pallas import tpu_sc as plsc

B = 4
C = 96
K = 1024
H = 512
W = 512
N = H * W

CG = 8
CH = C // CG
P = 4096
T = N // P
ROWS = P // W

_mesh = plsc.VectorSubcoreMesh(core_axis_name="c", subcore_axis_name="s")

_cp = pltpu.CompilerParams()
if "needs_layout_passes" in pltpu.CompilerParams.__dataclass_fields__:
    _cp = dataclasses.replace(_cp, needs_layout_passes=False)


def _unpool_body(pooled_hbm, spx_hbm, out_hbm, table_v,
                 idx_v, out_t, sem_i, sem_o):
    wid = lax.axis_index("c") * 16 + lax.axis_index("s")
    b = wid // CG
    cg = wid % CG

    tbl_off = (b * C + cg * CH) * K
    pltpu.sync_copy(pooled_hbm.at[pl.ds(tbl_off, CH * K)], table_v)

    idx_base = b * N

    def start_idx(t, buf):
        pltpu.async_copy(spx_hbm.at[pl.ds(idx_base + t * P, P)],
                         idx_v.at[buf], sem_i[buf])

    def item(t, buf):
        pltpu.make_async_copy(spx_hbm.at[pl.ds(idx_base, P)],
                              idx_v.at[buf], sem_i[buf]).wait()

        @pl.when(t >= 2)
        def _():
            pltpu.make_async_copy(
                out_t.at[buf],
                out_hbm.at[b, pl.ds(cg * CH, CH), pl.ds(0, ROWS), :],
                sem_o[buf]).wait()

        @plsc.parallel_loop(0, P, 16, unroll=2)
        def _vec(i):
            j = i // W
            col = i - j * W
            idx = idx_v[buf, pl.ds(i, 16)]
            for c in range(CH):
                out_t[buf, c, j, pl.ds(col, 16)] = plsc.load_gather(
                    table_v.at[pl.ds(c * K, K)], [idx])

        pltpu.async_copy(
            out_t.at[buf],
            out_hbm.at[b, pl.ds(cg * CH, CH), pl.ds(t * ROWS, ROWS), :],
            sem_o[buf])

        @pl.when(t + 2 < T)
        def _():
            start_idx(t + 2, buf)

    start_idx(0, 0)
    start_idx(1, 1)

    @pl.loop(0, T, step=2)
    def _tile(t):
        item(t, 0)
        item(t + 1, 1)

    for buf in range(2):
        pltpu.make_async_copy(
            out_t.at[buf],
            out_hbm.at[b, pl.ds(cg * CH, CH), pl.ds(0, ROWS), :],
            sem_o[buf]).wait()


@jax.jit
def kernel(pooled, spx):
    spx_i = spx.reshape(B * N).astype(jnp.int32)
    pooled_f = pooled.reshape(B * C * K)

    k = pl.kernel(
        _unpool_body,
        out_type=jax.ShapeDtypeStruct((B, C, H, W), jnp.float32),
        mesh=_mesh,
        compiler_params=_cp,
        scratch_types=[
            pltpu.VMEM((CH * K,), jnp.float32),
            pltpu.VMEM((2, P), jnp.int32),
            pltpu.VMEM((2, CH, ROWS, W), jnp.float32),
            [pltpu.SemaphoreType.DMA] * 2,
            [pltpu.SemaphoreType.DMA] * 2,
        ],
    )
    return k(pooled_f, spx_i)


# speedup vs baseline: 2.3267x; 1.0017x over previous
import dataclasses

import jax
import jax.numpy as jnp
from jax import lax
from jax.experimental import pallas as pl
from jax.experimental.pallas import tpu as pltpu
from jax.experimental.pallas import tpu_sc as plsc

B = 4
C = 96
K = 1024
H = 512
W = 512
N = H * W

CG = 8
CH = C // CG
P = 4096
T = N // P
ROWS = P // W

_mesh = plsc.VectorSubcoreMesh(core_axis_name="c", subcore_axis_name="s")

_cp = pltpu.CompilerParams()
if "needs_layout_passes" in pltpu.CompilerParams.__dataclass_fields__:
    _cp = dataclasses.replace(_cp, needs_layout_passes=False)


def _unpool_body(pooled_hbm, spx_hbm, out_hbm, table_v,
                 idx_v, out_t, sem_i, sem_o):
    wid = lax.axis_index("c") * 16 + lax.axis_index("s")
    b = wid // CG
    cg = wid % CG

    tbl_off = (b * C + cg * CH) * K
    pltpu.sync_copy(pooled_hbm.at[pl.ds(tbl_off, CH * K)], table_v)

    idx_base = b * N

    def start_idx(t, buf):
        pltpu.async_copy(spx_hbm.at[pl.ds(idx_base + t * P, P)],
                         idx_v.at[buf], sem_i[buf])

    def item(t, buf):
        pltpu.make_async_copy(spx_hbm.at[pl.ds(idx_base, P)],
                              idx_v.at[buf], sem_i[buf]).wait()

        @pl.when(t >= 2)
        def _():
            pltpu.make_async_copy(
                out_t.at[buf],
                out_hbm.at[b, pl.ds(cg * CH, CH), pl.ds(0, ROWS), :],
                sem_o[buf]).wait()

        @plsc.parallel_loop(0, P, 16, unroll=4)
        def _vec(i):
            j = i // W
            col = i - j * W
            idx = idx_v[buf, pl.ds(i, 16)]
            for c in range(CH):
                out_t[buf, c, j, pl.ds(col, 16)] = plsc.load_gather(
                    table_v.at[pl.ds(c * K, K)], [idx])

        pltpu.async_copy(
            out_t.at[buf],
            out_hbm.at[b, pl.ds(cg * CH, CH), pl.ds(t * ROWS, ROWS), :],
            sem_o[buf])

        @pl.when(t + 2 < T)
        def _():
            start_idx(t + 2, buf)

    start_idx(0, 0)
    start_idx(1, 1)

    @pl.loop(0, T, step=2)
    def _tile(t):
        item(t, 0)
        item(t + 1, 1)

    for buf in range(2):
        pltpu.make_async_copy(
            out_t.at[buf],
            out_hbm.at[b, pl.ds(cg * CH, CH), pl.ds(0, ROWS), :],
            sem_o[buf]).wait()


@jax.jit
def kernel(pooled, spx):
    spx_i = spx.reshape(B * N).astype(jnp.int32)
    pooled_f = pooled.reshape(B * C * K)

    k = pl.kernel(
        _unpool_body,
        out_type=jax.ShapeDtypeStruct((B, C, H, W), jnp.float32),
        mesh=_mesh,
        compiler_params=_cp,
        scratch_types=[
            pltpu.VMEM((CH * K,), jnp.float32),
            pltpu.VMEM((2, P), jnp.int32),
            pltpu.VMEM((2, CH, ROWS, W), jnp.float32),
            [pltpu.SemaphoreType.DMA] * 2,
            [pltpu.SemaphoreType.DMA] * 2,
        ],
    )
    return k(pooled_f, spx_i)
